# baseline (speedup 1.0000x reference)
import functools

import jax
import jax.numpy as jnp
from jax import lax
from jax.experimental import pallas as pl
from jax.experimental.pallas import tpu as pltpu
from jax.experimental.pallas import tpu_sc as plsc

N_USER = 25000
N_ITEM = 25000
N = N_USER + N_ITEM
N_HALF = N // 2
NNZ = 800000
EMB = 80
B = 4096
N_LAYER = 3
EMB_RATIO = 0.5

ROW_BLK = 2000
EBLK = 512
SUBBLK = 4
NNZ_PAD = ((NNZ + EBLK - 1) // EBLK) * EBLK
N_BLOCKS = NNZ_PAD // EBLK
PH = 4
N_OCT = N // (2 * PH)
WB_ROWS = 50
WB_CHUNKS = N_OCT // WB_ROWS
NSUB = 16


def _scalar(vec16, i):
    return lax.squeeze(lax.slice(vec16, (i,), (i + 1,)), (0,))


def _spmm_body(e_hbm, pki_hbm, pkv_hbm, sm_hbm, out_hbm,
               acc_sh, pbuf, vbuf, gathb, zbuf, smv, semp, semg, sems):
    c = lax.axis_index("c")
    s = lax.axis_index("s")

    pltpu.sync_copy(sm_hbm, smv)
    sm = smv[...]

    def _zrow(i, _):
        for f in range(EMB // 16):
            zbuf[i, pl.ds(f * 16, 16)] = jnp.zeros((16,), jnp.float32)
        return 0
    lax.fori_loop(0, WB_ROWS, _zrow, 0)
    nz = (WB_CHUNKS - s + NSUB - 1) // NSUB

    for phase in range(PH):
        lo = jnp.where(c == 0, _scalar(sm, 2 * phase),
                       _scalar(sm, 2 * (phase + PH)))
        hi = jnp.where(c == 0, _scalar(sm, 2 * phase + 1),
                       _scalar(sm, 2 * (phase + PH) + 1))
        rlo = (PH * c + phase) * N_OCT
        nblk = jnp.maximum(0, (hi - lo - s + NSUB - 1) // NSUB)

        def _zchunk(i, _):
            cidx = s + i * NSUB
            pltpu.sync_copy(zbuf, acc_sh.at[pl.ds(cidx * WB_ROWS, WB_ROWS)])
            return 0
        lax.fori_loop(0, nz, _zchunk, 0)
        plsc.subcore_barrier()

        def _blk_of(i):
            return jnp.minimum(lo + s + i * NSUB, N_BLOCKS - 1)

        def _stage(i):
            pltpu.async_copy(pki_hbm.at[_blk_of(i)], pbuf.at[i % 3], semp)
            pltpu.async_copy(pkv_hbm.at[_blk_of(i)], vbuf.at[i % 3], semp)

        def _wait_stage():
            pltpu.make_async_copy(pki_hbm.at[0], pbuf.at[0], semp).wait()
            pltpu.make_async_copy(pkv_hbm.at[0], vbuf.at[0], semp).wait()

        def _mask(i):
            d = i % 3
            for t in range(SUBBLK):
                for k in range(128 // 16):
                    sl = pl.ds(k * 16, 16)
                    r = pbuf[d, SUBBLK + t, sl]
                    v = vbuf[d, t, sl]
                    own = (r >= rlo) & (r < rlo + N_OCT)
                    vbuf[d, t, sl] = jnp.where(own, v, 0.0)
                    pbuf[d, SUBBLK + t, sl] = jnp.clip(r - rlo, 0, N_OCT - 1)

        def _gathers(i):
            for t in range(SUBBLK):
                pltpu.async_copy(e_hbm.at[pbuf.at[i % 3, t]],
                                 gathb.at[i % 2, t], semg)

        def _wait_gathers():
            for t in range(SUBBLK):
                pltpu.make_async_copy(e_hbm.at[pl.ds(0, 128)],
                                      gathb.at[0, 0], semg).wait()

        def _scale(i):
            d = i % 2
            dp = i % 3
            for t in range(SUBBLK):
                def _edge(e, _):
                    v16 = vbuf[dp, t, pl.ds((e // 16) * 16, 16)]
                    bval = lax.gather(
                        v16, jnp.full((16, 1), e % 16, jnp.int32),
                        dimension_numbers=lax.GatherDimensionNumbers(
                            offset_dims=(), collapsed_slice_dims=(0,),
                            start_index_map=(0,)),
                        slice_sizes=(1,),
                        mode=lax.GatherScatterMode.PROMISE_IN_BOUNDS)
                    for f in range(EMB // 16):
                        g = gathb[d, t, e, pl.ds(f * 16, 16)]
                        gathb[d, t, e, pl.ds(f * 16, 16)] = g * bval
                    return 0
                lax.fori_loop(0, 128, _edge, 0)

        def _scatters(i):
            for t in range(SUBBLK):
                pltpu.async_copy(gathb.at[i % 2, t],
                                 acc_sh.at[pbuf.at[i % 3, SUBBLK + t]],
                                 sems, add=True)

        def _wait_scatters():
            for t in range(SUBBLK):
                pltpu.make_async_copy(gathb.at[0, 0],
                                      acc_sh.at[pl.ds(0, 128)], sems).wait()

        @pl.when(nblk > 0)
        def _prologue():
            pltpu.sync_copy(pki_hbm.at[_blk_of(0)], pbuf.at[0])
            pltpu.sync_copy(pkv_hbm.at[_blk_of(0)], vbuf.at[0])
            _mask(0)
            _gathers(0)
            _stage(1)

        def _body(i, _):
            _wait_stage()
            _mask(i + 1)
            _wait_gathers()

            @pl.when(i > 0)
            def _():
                _wait_scatters()
            _gathers(i + 1)
            _scale(i)
            _scatters(i)
            _stage(i + 2)
            return 0
        lax.fori_loop(0, jnp.maximum(nblk - 1, 0), _body, 0)

        @pl.when(nblk > 0)
        def _epilogue():
            _wait_stage()
            _wait_gathers()

            @pl.when(nblk > 1)
            def _():
                _wait_scatters()
            last = nblk - 1
            _scale(last)
            _scatters(last)
            _wait_scatters()
        plsc.subcore_barrier()

        def _wb(i, _):
            cidx = s + i * NSUB
            pltpu.sync_copy(acc_sh.at[pl.ds(cidx * WB_ROWS, WB_ROWS)],
                            out_hbm.at[pl.ds(rlo + cidx * WB_ROWS, WB_ROWS)])
            return 0
        lax.fori_loop(0, nz, _wb, 0)


_spmm_call = functools.partial(
    pl.kernel,
    out_type=jax.ShapeDtypeStruct((N, EMB), jnp.float32),
    mesh=plsc.VectorSubcoreMesh(core_axis_name="c", subcore_axis_name="s"),
    compiler_params=pltpu.CompilerParams(use_tc_tiling_on_sc=False),
    scratch_types=[
        pltpu.VMEM_SHARED((N_OCT, EMB), jnp.float32),
        pltpu.VMEM((3, 2 * SUBBLK, 128), jnp.int32),
        pltpu.VMEM((3, SUBBLK, 128), jnp.float32),
        pltpu.VMEM((2, SUBBLK, 128, EMB), jnp.float32),
        pltpu.VMEM((WB_ROWS, EMB), jnp.float32),
        pltpu.VMEM((16,), jnp.int32),
        pltpu.SemaphoreType.DMA,
        pltpu.SemaphoreType.DMA,
        pltpu.SemaphoreType.DMA,
    ],
)(_spmm_body)


def _sc_spmm(E, packed_i, packed_v, sm):
    return _spmm_call(E, packed_i, packed_v, sm)


def _dense_layer_body(e_ref, l_ref, w1_ref, w2_ref, bias_ref, out_ref):
    e = e_ref[...]
    l = l_ref[...]
    acc = jnp.dot(l + e, w1_ref[...], preferred_element_type=jnp.float32)
    acc += jnp.dot(l * e, w2_ref[...], preferred_element_type=jnp.float32)
    acc += bias_ref[...]
    out_ref[...] = jnp.where(acc >= 0.0, acc, 0.2 * acc)


def _dense_layer(E, L_E, W1i, W2i, bias):
    grid = (N // ROW_BLK,)
    return pl.pallas_call(
        _dense_layer_body,
        grid=grid,
        in_specs=[
            pl.BlockSpec((ROW_BLK, EMB), lambda i: (i, 0)),
            pl.BlockSpec((ROW_BLK, EMB), lambda i: (i, 0)),
            pl.BlockSpec((EMB, EMB), lambda i: (0, 0)),
            pl.BlockSpec((EMB, EMB), lambda i: (0, 0)),
            pl.BlockSpec((1, EMB), lambda i: (0, 0)),
        ],
        out_specs=pl.BlockSpec((ROW_BLK, EMB), lambda i: (i, 0)),
        out_shape=jax.ShapeDtypeStruct((N, EMB), jnp.float32),
    )(E, L_E, W1i, W2i, bias.reshape(1, EMB))


def _norm_body(g_ref, out_ref):
    g = g_ref[...]
    nrm = jnp.sqrt(jnp.sum(g * g, axis=1, keepdims=True))
    out_ref[...] = g / jnp.maximum(nrm, 1e-12)


def _normalize_rows(G):
    m = G.shape[0]
    blk = 2048
    pad = (-m) % blk
    Gp = jnp.pad(G, ((0, pad), (0, 0)))
    out = pl.pallas_call(
        _norm_body,
        grid=((m + pad) // blk,),
        in_specs=[pl.BlockSpec((blk, EMB), lambda i: (i, 0))],
        out_specs=pl.BlockSpec((blk, EMB), lambda i: (i, 0)),
        out_shape=jax.ShapeDtypeStruct((m + pad, EMB), jnp.float32),
    )(Gp)
    return out[:m]


def kernel(user_table, item_table, age_table, sex_table, month_table, day_table,
           dow_table, W1, b1, W2, b2, lap_vals, u_id, age, sex, month, day, dow,
           pos_item, neg_item, lap_rows, lap_cols, year, node_flag):
    feats = jnp.concatenate([age_table[age], sex_table[sex], month_table[month],
                             day_table[day], dow_table[dow]], axis=1)
    upd = user_table[u_id] * (1.0 - EMB_RATIO) + feats * EMB_RATIO
    user_table = user_table.at[u_id].set(upd)
    E = jnp.concatenate([user_table, item_table], axis=0)

    pad = NNZ_PAD - NNZ
    cols3 = jnp.pad(lap_cols, (0, pad)).reshape(N_BLOCKS, SUBBLK, 128)
    rows3 = jnp.pad(lap_rows, (0, pad),
                    constant_values=N - 1).reshape(N_BLOCKS, SUBBLK, 128)
    packed_i = jnp.concatenate([cols3, rows3], axis=1)
    packed_v = jnp.pad(lap_vals, (0, pad)).reshape(N_BLOCKS, SUBBLK, 128)
    Sq = jnp.searchsorted(
        lap_rows, jnp.arange(N_OCT, N, N_OCT, jnp.int32)).astype(jnp.int32)
    lo_q = jnp.concatenate([jnp.zeros((1,), jnp.int32), Sq // EBLK])
    hi_q = jnp.concatenate([(Sq + EBLK - 1) // EBLK,
                            jnp.full((1,), N_BLOCKS, jnp.int32)])
    sm = jnp.stack([lo_q, hi_q], axis=1).reshape(2 * 2 * PH)

    raw = [E]
    for i in range(N_LAYER):
        L_E = _sc_spmm(E, packed_i, packed_v, sm)
        bias = 2.0 * b1[i] + b2[i]
        E = _dense_layer(E, L_E, W1[i], W2[i], bias)
        raw.append(E)

    cat_idx = jnp.concatenate([u_id, N_USER + pos_item, N_USER + neg_item])
    g0 = raw[0][cat_idx]
    gs = [g0] + [_normalize_rows(raw[i + 1][cat_idx]) for i in range(N_LAYER)]
    allg = jnp.concatenate(gs, axis=1)
    return (allg[:B], allg[B:2 * B], allg[2 * B:])


# speedup vs baseline: 1.2464x; 1.2464x over previous
import functools

import jax
import jax.numpy as jnp
from jax import lax
from jax.experimental import pallas as pl
from jax.experimental.pallas import tpu as pltpu
from jax.experimental.pallas import tpu_sc as plsc

N_USER = 25000
N_ITEM = 25000
N = N_USER + N_ITEM
N_HALF = N // 2
NNZ = 800000
EMB = 80
B = 4096
N_LAYER = 3
EMB_RATIO = 0.5

ROW_BLK = 2000
EBLK = 512
SUBBLK = 4
NNZ_PAD = ((NNZ + EBLK - 1) // EBLK) * EBLK
N_BLOCKS = NNZ_PAD // EBLK
PH = 4
N_OCT = N // (2 * PH)
WB_ROWS = 50
WB_CHUNKS = N_OCT // WB_ROWS
NSUB = 16


def _scalar(vec16, i):
    return lax.squeeze(lax.slice(vec16, (i,), (i + 1,)), (0,))


def _bcast(vec16, i):
    return lax.gather(
        vec16, jnp.full((16, 1), i, jnp.int32),
        dimension_numbers=lax.GatherDimensionNumbers(
            offset_dims=(), collapsed_slice_dims=(0,), start_index_map=(0,)),
        slice_sizes=(1,),
        mode=lax.GatherScatterMode.PROMISE_IN_BOUNDS)


def _spmm_body(e_hbm, pki_hbm, pkv_hbm, sm_hbm, out_hbm,
               acc_sh, pbuf, vbuf, gathb, zbuf, smv, sms, semp, semg, sems):
    c = lax.axis_index("c")
    s = lax.axis_index("s")

    pltpu.sync_copy(sm_hbm, smv)
    sm = smv[...]
    for j in range(2 * 2 * PH):
        sms[j] = _scalar(sm, j)

    def _zrow(i, _):
        for f in range(EMB // 16):
            zbuf[i, pl.ds(f * 16, 16)] = jnp.zeros((16,), jnp.float32)
        return 0
    lax.fori_loop(0, WB_ROWS, _zrow, 0)
    nz = (WB_CHUNKS - s + NSUB - 1) // NSUB

    def _phase(p, _):
        o = PH * c + p
        lo = sms[2 * o]
        hi = sms[2 * o + 1]
        rlo = o * N_OCT
        nblk = jnp.maximum(0, (hi - lo - s + NSUB - 1) // NSUB)

        def _zchunk(i, _):
            cidx = s + i * NSUB
            pltpu.sync_copy(zbuf, acc_sh.at[pl.ds(cidx * WB_ROWS, WB_ROWS)])
            return 0
        lax.fori_loop(0, nz, _zchunk, 0)
        plsc.subcore_barrier()

        def _blk_of(i):
            return jnp.minimum(lo + s + i * NSUB, N_BLOCKS - 1)

        def _stage(i):
            pltpu.async_copy(pki_hbm.at[_blk_of(i)], pbuf.at[i % 3], semp)
            pltpu.async_copy(pkv_hbm.at[_blk_of(i)], vbuf.at[i % 3], semp)

        def _wait_stage():
            pltpu.make_async_copy(pki_hbm.at[0], pbuf.at[0], semp).wait()
            pltpu.make_async_copy(pkv_hbm.at[0], vbuf.at[0], semp).wait()

        def _mask(i):
            d = i % 3
            for t in range(SUBBLK):
                for k in range(128 // 16):
                    sl = pl.ds(k * 16, 16)
                    r = pbuf[d, SUBBLK + t, sl]
                    v = vbuf[d, t, sl]
                    own = (r >= rlo) & (r < rlo + N_OCT)
                    vbuf[d, t, sl] = jnp.where(own, v, 0.0)
                    pbuf[d, SUBBLK + t, sl] = jnp.clip(r - rlo, 0, N_OCT - 1)

        def _gathers(i):
            for t in range(SUBBLK):
                pltpu.async_copy(e_hbm.at[pbuf.at[i % 3, t]],
                                 gathb.at[i % 2, t], semg)

        def _wait_gathers():
            for t in range(SUBBLK):
                pltpu.make_async_copy(e_hbm.at[pl.ds(0, 128)],
                                      gathb.at[0, 0], semg).wait()

        def _scale(i):
            d = i % 2
            dp = i % 3
            for t in range(SUBBLK):
                @plsc.parallel_loop(0, 128 // 16, unroll=2)
                def _grp(g):
                    v16 = vbuf[dp, t, pl.ds(g * 16, 16)]
                    for j in range(16):
                        bval = _bcast(v16, j)
                        e = g * 16 + j
                        for f in range(EMB // 16):
                            x = gathb[d, t, e, pl.ds(f * 16, 16)]
                            gathb[d, t, e, pl.ds(f * 16, 16)] = x * bval

        def _scatters(i):
            for t in range(SUBBLK):
                pltpu.async_copy(gathb.at[i % 2, t],
                                 acc_sh.at[pbuf.at[i % 3, SUBBLK + t]],
                                 sems, add=True)

        def _wait_scatters():
            for t in range(SUBBLK):
                pltpu.make_async_copy(gathb.at[0, 0],
                                      acc_sh.at[pl.ds(0, 128)], sems).wait()

        @pl.when(nblk > 0)
        def _prologue():
            pltpu.sync_copy(pki_hbm.at[_blk_of(0)], pbuf.at[0])
            pltpu.sync_copy(pkv_hbm.at[_blk_of(0)], vbuf.at[0])
            _mask(0)
            _gathers(0)

            @pl.when(nblk > 1)
            def _():
                _stage(1)

        def _body(i, _):
            @pl.when(i + 1 < nblk)
            def _():
                _wait_stage()
                _mask(i + 1)
            _wait_gathers()

            @pl.when(i > 0)
            def _():
                _wait_scatters()

            @pl.when(i + 1 < nblk)
            def _():
                _gathers(i + 1)
            _scale(i)
            _scatters(i)

            @pl.when(i + 2 < nblk)
            def _():
                _stage(i + 2)
            return 0
        lax.fori_loop(0, nblk, _body, 0)

        @pl.when(nblk > 0)
        def _drain():
            _wait_scatters()
        plsc.subcore_barrier()

        def _wb(i, _):
            cidx = s + i * NSUB
            pltpu.sync_copy(acc_sh.at[pl.ds(cidx * WB_ROWS, WB_ROWS)],
                            out_hbm.at[pl.ds(rlo + cidx * WB_ROWS, WB_ROWS)])
            return 0
        lax.fori_loop(0, nz, _wb, 0)
        return 0

    lax.fori_loop(0, PH, _phase, 0)


_spmm_call = functools.partial(
    pl.kernel,
    out_type=jax.ShapeDtypeStruct((N, EMB), jnp.float32),
    mesh=plsc.VectorSubcoreMesh(core_axis_name="c", subcore_axis_name="s"),
    compiler_params=pltpu.CompilerParams(use_tc_tiling_on_sc=False),
    scratch_types=[
        pltpu.VMEM_SHARED((N_OCT, EMB), jnp.float32),
        pltpu.VMEM((3, 2 * SUBBLK, 128), jnp.int32),
        pltpu.VMEM((3, SUBBLK, 128), jnp.float32),
        pltpu.VMEM((2, SUBBLK, 128, EMB), jnp.float32),
        pltpu.VMEM((WB_ROWS, EMB), jnp.float32),
        pltpu.VMEM((16,), jnp.int32),
        pltpu.SMEM((16,), jnp.int32),
        pltpu.SemaphoreType.DMA,
        pltpu.SemaphoreType.DMA,
        pltpu.SemaphoreType.DMA,
    ],
)(_spmm_body)


def _sc_spmm(E, packed_i, packed_v, sm):
    return _spmm_call(E, packed_i, packed_v, sm)


def _dense_layer_body(e_ref, l_ref, w1_ref, w2_ref, bias_ref, out_ref):
    e = e_ref[...]
    l = l_ref[...]
    acc = jnp.dot(l + e, w1_ref[...], preferred_element_type=jnp.float32)
    acc += jnp.dot(l * e, w2_ref[...], preferred_element_type=jnp.float32)
    acc += bias_ref[...]
    out_ref[...] = jnp.where(acc >= 0.0, acc, 0.2 * acc)


def _dense_layer(E, L_E, W1i, W2i, bias):
    grid = (N // ROW_BLK,)
    return pl.pallas_call(
        _dense_layer_body,
        grid=grid,
        in_specs=[
            pl.BlockSpec((ROW_BLK, EMB), lambda i: (i, 0)),
            pl.BlockSpec((ROW_BLK, EMB), lambda i: (i, 0)),
            pl.BlockSpec((EMB, EMB), lambda i: (0, 0)),
            pl.BlockSpec((EMB, EMB), lambda i: (0, 0)),
            pl.BlockSpec((1, EMB), lambda i: (0, 0)),
        ],
        out_specs=pl.BlockSpec((ROW_BLK, EMB), lambda i: (i, 0)),
        out_shape=jax.ShapeDtypeStruct((N, EMB), jnp.float32),
    )(E, L_E, W1i, W2i, bias.reshape(1, EMB))


def _norm_body(g_ref, out_ref):
    g = g_ref[...]
    nrm = jnp.sqrt(jnp.sum(g * g, axis=1, keepdims=True))
    out_ref[...] = g / jnp.maximum(nrm, 1e-12)


def _normalize_rows(G):
    m = G.shape[0]
    blk = 2048
    pad = (-m) % blk
    Gp = jnp.pad(G, ((0, pad), (0, 0)))
    out = pl.pallas_call(
        _norm_body,
        grid=((m + pad) // blk,),
        in_specs=[pl.BlockSpec((blk, EMB), lambda i: (i, 0))],
        out_specs=pl.BlockSpec((blk, EMB), lambda i: (i, 0)),
        out_shape=jax.ShapeDtypeStruct((m + pad, EMB), jnp.float32),
    )(Gp)
    return out[:m]


def kernel(user_table, item_table, age_table, sex_table, month_table, day_table,
           dow_table, W1, b1, W2, b2, lap_vals, u_id, age, sex, month, day, dow,
           pos_item, neg_item, lap_rows, lap_cols, year, node_flag):
    feats = jnp.concatenate([age_table[age], sex_table[sex], month_table[month],
                             day_table[day], dow_table[dow]], axis=1)
    upd = user_table[u_id] * (1.0 - EMB_RATIO) + feats * EMB_RATIO
    user_table = user_table.at[u_id].set(upd)
    E = jnp.concatenate([user_table, item_table], axis=0)

    pad = NNZ_PAD - NNZ
    cols3 = jnp.pad(lap_cols, (0, pad)).reshape(N_BLOCKS, SUBBLK, 128)
    rows3 = jnp.pad(lap_rows, (0, pad),
                    constant_values=N - 1).reshape(N_BLOCKS, SUBBLK, 128)
    packed_i = jnp.concatenate([cols3, rows3], axis=1)
    packed_v = jnp.pad(lap_vals, (0, pad)).reshape(N_BLOCKS, SUBBLK, 128)
    Sq = jnp.searchsorted(
        lap_rows, jnp.arange(N_OCT, N, N_OCT, jnp.int32)).astype(jnp.int32)
    lo_q = jnp.concatenate([jnp.zeros((1,), jnp.int32), Sq // EBLK])
    hi_q = jnp.concatenate([(Sq + EBLK - 1) // EBLK,
                            jnp.full((1,), N_BLOCKS, jnp.int32)])
    sm = jnp.stack([lo_q, hi_q], axis=1).reshape(2 * 2 * PH)

    raw = [E]
    for i in range(N_LAYER):
        L_E = _sc_spmm(E, packed_i, packed_v, sm)
        bias = 2.0 * b1[i] + b2[i]
        E = _dense_layer(E, L_E, W1[i], W2[i], bias)
        raw.append(E)

    cat_idx = jnp.concatenate([u_id, N_USER + pos_item, N_USER + neg_item])
    g0 = raw[0][cat_idx]
    gs = [g0] + [_normalize_rows(raw[i + 1][cat_idx]) for i in range(N_LAYER)]
    allg = jnp.concatenate(gs, axis=1)
    return (allg[:B], allg[B:2 * B], allg[2 * B:])
